# baseline (speedup 1.0000x reference)
import jax
import jax.numpy as jnp
from jax.experimental import pallas as pl
from jax.experimental.pallas import tpu as pltpu


def _pick_block(n, cap):
    best = 8
    for b in range(8, cap + 1, 8):
        if n % b == 0:
            best = b
    return best


def _encode_proj_body(ids_ref, emb_ref, temb_ref, w1_ref, out_ref):
    ids = ids_ref[...]
    nt = temb_ref.shape[0]
    onehot = (ids == jax.lax.broadcasted_iota(jnp.int32, (ids.shape[0], nt), 1))
    x = emb_ref[...] + jnp.dot(onehot.astype(jnp.float32), temb_ref[...],
                               preferred_element_type=jnp.float32)
    out_ref[...] = jnp.dot(x, w1_ref[...], preferred_element_type=jnp.float32)


def _spmm_relu_proj_body(adj_ref, y_ref, b1_ref, w2_ref, out_ref):
    t = jnp.dot(adj_ref[...].astype(jnp.bfloat16), y_ref[...].astype(jnp.bfloat16),
                preferred_element_type=jnp.float32)
    h = jnp.maximum(t + b1_ref[...], 0.0)
    out_ref[...] = jnp.dot(h, w2_ref[...], preferred_element_type=jnp.float32)


def _spmm_bias_body(adj_ref, y_ref, b2_ref, out_ref):
    out_ref[...] = jnp.dot(adj_ref[...].astype(jnp.bfloat16),
                           y_ref[...].astype(jnp.bfloat16),
                           preferred_element_type=jnp.float32) + b2_ref[...]


def kernel(node_type_ids, adj, node_emb, type_emb, W1, b1, W2, b2):
    N, H = node_emb.shape
    E = W2.shape[1]
    T = type_emb.shape[0]
    ids2 = node_type_ids.reshape(N, 1)
    b1r = b1.reshape(1, H)
    b2r = b2.reshape(1, E)

    TB = _pick_block(N, 2048)
    y1 = pl.pallas_call(
        _encode_proj_body,
        grid=(N // TB,),
        in_specs=[
            pl.BlockSpec((TB, 1), lambda i: (i, 0)),
            pl.BlockSpec((TB, H), lambda i: (i, 0)),
            pl.BlockSpec((T, H), lambda i: (0, 0)),
            pl.BlockSpec((H, H), lambda i: (0, 0)),
        ],
        out_specs=pl.BlockSpec((TB, H), lambda i: (i, 0)),
        out_shape=jax.ShapeDtypeStruct((N, H), jnp.float32),
        compiler_params=pltpu.CompilerParams(
            dimension_semantics=("arbitrary",)),
    )(ids2, node_emb, type_emb, W1)

    TI = _pick_block(N, 512)
    y2 = pl.pallas_call(
        _spmm_relu_proj_body,
        grid=(N // TI,),
        in_specs=[
            pl.BlockSpec((TI, N), lambda i: (i, 0)),
            pl.BlockSpec((N, H), lambda i: (0, 0)),
            pl.BlockSpec((1, H), lambda i: (0, 0)),
            pl.BlockSpec((H, E), lambda i: (0, 0)),
        ],
        out_specs=pl.BlockSpec((TI, E), lambda i: (i, 0)),
        out_shape=jax.ShapeDtypeStruct((N, E), jnp.float32),
        compiler_params=pltpu.CompilerParams(
            dimension_semantics=("arbitrary",)),
    )(adj, y1, b1r, W2)

    z = pl.pallas_call(
        _spmm_bias_body,
        grid=(N // TI,),
        in_specs=[
            pl.BlockSpec((TI, N), lambda i: (i, 0)),
            pl.BlockSpec((N, E), lambda i: (0, 0)),
            pl.BlockSpec((1, E), lambda i: (0, 0)),
        ],
        out_specs=pl.BlockSpec((TI, E), lambda i: (i, 0)),
        out_shape=jax.ShapeDtypeStruct((N, E), jnp.float32),
        compiler_params=pltpu.CompilerParams(
            dimension_semantics=("arbitrary",)),
    )(adj, y2, b2r)
    return z


# speedup vs baseline: 1.0001x; 1.0001x over previous
import jax
import jax.numpy as jnp
from jax.experimental import pallas as pl
from jax.experimental.pallas import tpu as pltpu


def _pick_block(n, cap):
    best = 8
    for b in range(8, cap + 1, 8):
        if n % b == 0:
            best = b
    return best


def _encode_proj_body(ids_ref, emb_ref, temb_ref, w1_ref, out_ref):
    ids = ids_ref[...]
    nt = temb_ref.shape[0]
    onehot = (ids == jax.lax.broadcasted_iota(jnp.int32, (ids.shape[0], nt), 1))
    x = emb_ref[...] + jnp.dot(onehot.astype(jnp.float32), temb_ref[...],
                               preferred_element_type=jnp.float32)
    out_ref[...] = jnp.dot(x, w1_ref[...], preferred_element_type=jnp.float32)


def _spmm_relu_proj_body(adj_ref, y_ref, b1_ref, w2_ref, out_ref):
    t = jnp.dot(adj_ref[...], y_ref[...], preferred_element_type=jnp.float32)
    h = jnp.maximum(t + b1_ref[...], 0.0)
    out_ref[...] = jnp.dot(h, w2_ref[...], preferred_element_type=jnp.float32)


def _spmm_bias_body(adj_ref, y_ref, b2_ref, out_ref):
    out_ref[...] = jnp.dot(adj_ref[...], y_ref[...],
                           preferred_element_type=jnp.float32) + b2_ref[...]


def kernel(node_type_ids, adj, node_emb, type_emb, W1, b1, W2, b2):
    N, H = node_emb.shape
    E = W2.shape[1]
    T = type_emb.shape[0]
    ids2 = node_type_ids.reshape(N, 1)
    b1r = b1.reshape(1, H)
    b2r = b2.reshape(1, E)

    TB = _pick_block(N, 2048)
    y1 = pl.pallas_call(
        _encode_proj_body,
        grid=(N // TB,),
        in_specs=[
            pl.BlockSpec((TB, 1), lambda i: (i, 0)),
            pl.BlockSpec((TB, H), lambda i: (i, 0)),
            pl.BlockSpec((T, H), lambda i: (0, 0)),
            pl.BlockSpec((H, H), lambda i: (0, 0)),
        ],
        out_specs=pl.BlockSpec((TB, H), lambda i: (i, 0)),
        out_shape=jax.ShapeDtypeStruct((N, H), jnp.float32),
        compiler_params=pltpu.CompilerParams(
            dimension_semantics=("parallel",)),
    )(ids2, node_emb, type_emb, W1)

    TI = _pick_block(N, 512)
    y2 = pl.pallas_call(
        _spmm_relu_proj_body,
        grid=(N // TI,),
        in_specs=[
            pl.BlockSpec((TI, N), lambda i: (i, 0)),
            pl.BlockSpec((N, H), lambda i: (0, 0)),
            pl.BlockSpec((1, H), lambda i: (0, 0)),
            pl.BlockSpec((H, E), lambda i: (0, 0)),
        ],
        out_specs=pl.BlockSpec((TI, E), lambda i: (i, 0)),
        out_shape=jax.ShapeDtypeStruct((N, E), jnp.float32),
        compiler_params=pltpu.CompilerParams(
            dimension_semantics=("parallel",)),
    )(adj, y1, b1r, W2)

    z = pl.pallas_call(
        _spmm_bias_body,
        grid=(N // TI,),
        in_specs=[
            pl.BlockSpec((TI, N), lambda i: (i, 0)),
            pl.BlockSpec((N, E), lambda i: (0, 0)),
            pl.BlockSpec((1, E), lambda i: (0, 0)),
        ],
        out_specs=pl.BlockSpec((TI, E), lambda i: (i, 0)),
        out_shape=jax.ShapeDtypeStruct((N, E), jnp.float32),
        compiler_params=pltpu.CompilerParams(
            dimension_semantics=("parallel",)),
    )(adj, y2, b2r)
    return z


# speedup vs baseline: 1.0151x; 1.0150x over previous
import jax
import jax.numpy as jnp
from jax.experimental import pallas as pl
from jax.experimental.pallas import tpu as pltpu


def _pick_block(n, cap):
    best = 8
    for b in range(8, cap + 1, 8):
        if n % b == 0:
            best = b
    return best


def _fused_gcn_body(ids_ref, emb_ref, temb_ref, w1_ref, b1_ref, w2_ref,
                    b2_ref, adj_ref, out_ref, y1_scr, y2_scr):
    p = pl.program_id(0)
    i = pl.program_id(1)
    ti = adj_ref.shape[0]

    @pl.when((p == 0) & (i == 0))
    def _encode():
        ids = ids_ref[...]
        nt = temb_ref.shape[0]
        onehot = (ids == jax.lax.broadcasted_iota(
            jnp.int32, (ids.shape[0], nt), 1))
        x = emb_ref[...] + jnp.dot(onehot.astype(jnp.float32), temb_ref[...],
                                   preferred_element_type=jnp.float32)
        y1_scr[...] = jnp.dot(x, w1_ref[...],
                              preferred_element_type=jnp.float32)

    @pl.when(p == 0)
    def _layer1():
        t = jnp.dot(adj_ref[...], y1_scr[...],
                    preferred_element_type=jnp.float32)
        h = jnp.maximum(t + b1_ref[...], 0.0)
        y2_blk = jnp.dot(h, w2_ref[...], preferred_element_type=jnp.float32)
        y2_scr[pl.ds(i * ti, ti), :] = y2_blk
        out_ref[0, :, :] = y2_blk

    @pl.when(p == 1)
    def _layer2():
        out_ref[0, :, :] = jnp.dot(adj_ref[...], y2_scr[...],
                                   preferred_element_type=jnp.float32) + b2_ref[...]


def kernel(node_type_ids, adj, node_emb, type_emb, W1, b1, W2, b2):
    N, H = node_emb.shape
    E = W2.shape[1]
    T = type_emb.shape[0]
    ids2 = node_type_ids.reshape(N, 1)
    b1r = b1.reshape(1, H)
    b2r = b2.reshape(1, E)

    TI = _pick_block(N, 256)
    full = lambda p, i: (0, 0)
    z = pl.pallas_call(
        _fused_gcn_body,
        grid=(2, N // TI),
        in_specs=[
            pl.BlockSpec((N, 1), full),
            pl.BlockSpec((N, H), full),
            pl.BlockSpec((T, H), full),
            pl.BlockSpec((H, H), full),
            pl.BlockSpec((1, H), full),
            pl.BlockSpec((H, E), full),
            pl.BlockSpec((1, E), full),
            pl.BlockSpec((TI, N), lambda p, i: (i, 0)),
        ],
        out_specs=pl.BlockSpec((1, TI, E), lambda p, i: (p, i, 0)),
        out_shape=jax.ShapeDtypeStruct((2, N, E), jnp.float32),
        scratch_shapes=[
            pltpu.VMEM((N, H), jnp.float32),
            pltpu.VMEM((N, E), jnp.float32),
        ],
        compiler_params=pltpu.CompilerParams(
            dimension_semantics=("arbitrary", "arbitrary")),
    )(ids2, node_emb, type_emb, W1, b1r, W2, b2r, adj)
    return z[1]


# speedup vs baseline: 1.0429x; 1.0274x over previous
import jax
import jax.numpy as jnp
from jax.experimental import pallas as pl
from jax.experimental.pallas import tpu as pltpu


def _pick_block(n, cap):
    best = 8
    for b in range(8, cap + 1, 8):
        if n % b == 0:
            best = b
    return best


def _fused_gcn_body(ids_ref, emb_ref, temb_ref, w1_ref, b1_ref, w2_ref,
                    b2_ref, adj_ref, out_ref, y1_scr, y2_scr):
    p = pl.program_id(0)
    i = pl.program_id(1)
    ti = adj_ref.shape[0]

    @pl.when((p == 0) & (i == 0))
    def _encode():
        ids = ids_ref[...]
        nt = temb_ref.shape[0]
        onehot = (ids == jax.lax.broadcasted_iota(
            jnp.int32, (ids.shape[0], nt), 1))
        x = emb_ref[...] + jnp.dot(onehot.astype(jnp.float32), temb_ref[...],
                                   preferred_element_type=jnp.float32)
        y1_scr[...] = jnp.dot(x, w1_ref[...],
                              preferred_element_type=jnp.float32)

    @pl.when(p == 0)
    def _layer1():
        t = jnp.dot(adj_ref[...], y1_scr[...],
                    preferred_element_type=jnp.float32)
        h = jnp.maximum(t + b1_ref[...], 0.0)
        y2_blk = jnp.dot(h, w2_ref[...], preferred_element_type=jnp.float32)
        y2_scr[pl.ds(i * ti, ti), :] = y2_blk
        out_ref[0, :, :] = y2_blk

    @pl.when(p == 1)
    def _layer2():
        out_ref[0, :, :] = jnp.dot(adj_ref[...], y2_scr[...],
                                   preferred_element_type=jnp.float32) + b2_ref[...]


def kernel(node_type_ids, adj, node_emb, type_emb, W1, b1, W2, b2):
    N, H = node_emb.shape
    E = W2.shape[1]
    T = type_emb.shape[0]
    ids2 = node_type_ids.reshape(N, 1)
    b1r = b1.reshape(1, H)
    b2r = b2.reshape(1, E)

    TI = _pick_block(N, 512)
    full = lambda p, i: (0, 0)
    z = pl.pallas_call(
        _fused_gcn_body,
        grid=(2, N // TI),
        in_specs=[
            pl.BlockSpec((N, 1), full),
            pl.BlockSpec((N, H), full),
            pl.BlockSpec((T, H), full),
            pl.BlockSpec((H, H), full),
            pl.BlockSpec((1, H), full),
            pl.BlockSpec((H, E), full),
            pl.BlockSpec((1, E), full),
            pl.BlockSpec((TI, N), lambda p, i: (i, 0)),
        ],
        out_specs=pl.BlockSpec((1, TI, E), lambda p, i: (p, i, 0)),
        out_shape=jax.ShapeDtypeStruct((2, N, E), jnp.float32),
        scratch_shapes=[
            pltpu.VMEM((N, H), jnp.float32),
            pltpu.VMEM((N, E), jnp.float32),
        ],
        compiler_params=pltpu.CompilerParams(
            dimension_semantics=("arbitrary", "arbitrary"),
            vmem_limit_bytes=128 * 1024 * 1024),
    )(ids2, node_emb, type_emb, W1, b1r, W2, b2r, adj)
    return z[1]
